# baseline (speedup 1.0000x reference)
import jax
import jax.numpy as jnp
from jax.experimental import pallas as pl
from jax.experimental.pallas import tpu as pltpu

_TN = 2048


def _edge_dot_kernel(e1_ref, e2_ref, o_ref):
    prod = e1_ref[...].astype(jnp.float32) * e2_ref[...].astype(jnp.float32)
    ones = jnp.ones((1, prod.shape[0]), dtype=jnp.float32)
    o_ref[...] = jax.lax.dot_general(
        ones, prod, (((1,), (0,)), ((), ())),
        preferred_element_type=jnp.float32,
    ).astype(o_ref.dtype)


def kernel(embeds1, embeds2):
    D, N = embeds1.shape
    assert embeds2.shape == (D, N)
    out_dtype = jnp.result_type(embeds1.dtype, embeds2.dtype)

    tn = min(_TN, N)
    n_blocks = pl.cdiv(N, tn)

    out = pl.pallas_call(
        _edge_dot_kernel,
        out_shape=jax.ShapeDtypeStruct((1, N), out_dtype),
        grid=(n_blocks,),
        in_specs=[
            pl.BlockSpec((D, tn), lambda i: (0, i)),
            pl.BlockSpec((D, tn), lambda i: (0, i)),
        ],
        out_specs=pl.BlockSpec((1, tn), lambda i: (0, i)),
        compiler_params=pltpu.CompilerParams(
            dimension_semantics=("parallel",),
            vmem_limit_bytes=48 * 1024 * 1024,
        ),
    )(embeds1, embeds2)
    return out[0]
